# baseline (speedup 1.0000x reference)
import jax
import jax.numpy as jnp
from jax import lax
from jax.experimental import pallas as pl
from jax.experimental.pallas import tpu as pltpu
from jax.experimental.pallas import tpu_sc as plsc

N = 10000
E = 320000
F_IN = 128
HID = 128
NC = 16

N_PAD = 10240
NR = N_PAD // 16
C = 256
E_TOT = E + N
N_CHUNKS = -(-E_TOT // (32 * C))
EPT = N_CHUNKS * C
E_PAD = 32 * EPT
DEN_W = 16

_mesh = plsc.VectorSubcoreMesh(core_axis_name="c", subcore_axis_name="s",
                               num_cores=2, num_subcores=16)
_sc_params = pltpu.CompilerParams(needs_layout_passes=False,
                                  use_tc_tiling_on_sc=False)


def _seg_max_update(m_loc, dvec, lg):
    def cond(state):
        it, go = state
        return jnp.logical_and(go, it < 16)

    def step(state):
        it, _ = state
        cur = plsc.load_gather(m_loc, [dvec])
        need = lg > cur
        plsc.store_scatter(m_loc, [dvec], jnp.maximum(cur, lg), mask=need)
        cur2 = plsc.load_gather(m_loc, [dvec])
        return it + 1, jnp.any(lg > cur2)

    lax.while_loop(cond, step, (jnp.int32(0), jnp.bool_(True)))


def _make_phase1(D):
    KG = D // 16

    def body(xl_hbm, xr_hbm, src_hbm, dst_hbm, a_hbm,
             logits_hbm, m_hbm, mpart_hbm,
             a_v, sidx, didx, U, V, lbuf, m_loc, mbuf, mred, P, sem):
        cid = lax.axis_index("c")
        tid = lax.axis_index("s")
        wid = cid * 16 + tid
        iota = lax.iota(jnp.int32, 16)
        pltpu.sync_copy(a_hbm, a_v)
        neg = jnp.full((16,), -3e38, jnp.float32)

        def zi(i, _):
            m_loc[pl.ds(i * 16, 16)] = neg
            return 0
        lax.fori_loop(0, N_PAD // 16, zi, 0)

        def chunk(nc, _):
            base = wid * EPT + nc * C
            pltpu.sync_copy(src_hbm.at[pl.ds(base, C)], sidx)
            pltpu.sync_copy(dst_hbm.at[pl.ds(base, C)], didx)
            pltpu.async_copy(xl_hbm.at[sidx], U, sem).wait()
            pltpu.async_copy(xr_hbm.at[didx], V, sem).wait()

            def group(g, _):
                def edge(j2, _):
                    for p in range(2):
                        jl = j2 * 2 + p
                        e = g * 16 + jl
                        acc = jnp.zeros((16,), jnp.float32)
                        for k in range(KG):
                            u = U[e, pl.ds(k * 16, 16)]
                            v = V[e, pl.ds(k * 16, 16)]
                            z = u + v
                            t = jnp.maximum(z, 0.2 * z)
                            acc = acc + t * a_v[pl.ds(k * 16, 16)]
                        P[jl, pl.ds(0, 16)] = acc
                    return 0
                lax.fori_loop(0, 8, edge, 0)
                lg = jnp.zeros((16,), jnp.float32)
                for j in range(16):
                    lg = lg + plsc.load_gather(
                        P, [iota, jnp.full((16,), j, jnp.int32)])
                lbuf[pl.ds(g * 16, 16)] = lg
                dvec = didx[pl.ds(g * 16, 16)]
                _seg_max_update(m_loc, dvec, lg)
                return 0
            lax.fori_loop(0, C // 16, group, 0)
            pltpu.sync_copy(lbuf, logits_hbm.at[pl.ds(base, C)])
            return 0
        lax.fori_loop(0, N_CHUNKS, chunk, 0)

        pltpu.sync_copy(m_loc, mpart_hbm.at[wid])
        plsc.subcore_barrier()
        row0 = tid * NR
        pltpu.sync_copy(mpart_hbm.at[pl.ds(cid * 16, 16), pl.ds(row0, NR)],
                        mbuf)

        def red(i, _):
            mv = mbuf[0, pl.ds(i * 16, 16)]
            for t in range(1, 16):
                mv = jnp.maximum(mv, mbuf[t, pl.ds(i * 16, 16)])
            mred[pl.ds(i * 16, 16)] = mv
            return 0
        lax.fori_loop(0, NR // 16, red, 0)
        pltpu.sync_copy(mred, m_hbm.at[cid, pl.ds(row0, NR)])

    return pl.kernel(
        body,
        out_type=[jax.ShapeDtypeStruct((E_PAD,), jnp.float32),
                  jax.ShapeDtypeStruct((2, N_PAD), jnp.float32),
                  jax.ShapeDtypeStruct((32, N_PAD), jnp.float32)],
        mesh=_mesh,
        compiler_params=_sc_params,
        scratch_types=[
            pltpu.VMEM((D,), jnp.float32),
            pltpu.VMEM((C,), jnp.int32),
            pltpu.VMEM((C,), jnp.int32),
            pltpu.VMEM((C, D), jnp.float32),
            pltpu.VMEM((C, D), jnp.float32),
            pltpu.VMEM((C,), jnp.float32),
            pltpu.VMEM((N_PAD,), jnp.float32),
            pltpu.VMEM((16, NR), jnp.float32),
            pltpu.VMEM((NR,), jnp.float32),
            pltpu.VMEM((16, 16), jnp.float32),
            pltpu.SemaphoreType.DMA,
        ],
    )


def _make_phase2(D):
    KG = D // 16

    def body(xt_hbm, src_hbm, dst_hbm, logits_hbm, mg_hbm,
             acc_hbm, den_hbm,
             sidx, didx, lbuf, U, den_b, m_loc, zbuf, zbuf2, ubuf,
             acc_sh, den_sh, sem):
        cid = lax.axis_index("c")
        tid = lax.axis_index("s")
        wid = cid * 16 + tid
        iota = lax.iota(jnp.int32, 16)
        zeros16 = jnp.zeros((16,), jnp.float32)

        pltpu.sync_copy(mg_hbm, m_loc)

        def zb(r, _):
            for k in range(KG):
                zbuf[r, pl.ds(k * 16, 16)] = zeros16
            zbuf2[r, pl.ds(0, 16)] = zeros16
            return 0
        lax.fori_loop(0, 8, zb, 0)

        def zd(r, _):
            den_b[r, pl.ds(0, 16)] = zeros16
            return 0
        lax.fori_loop(0, C, zd, 0)
        row0 = tid * NR
        for i in range(NR // 8):
            pltpu.sync_copy(zbuf, acc_sh.at[pl.ds(row0 + i * 8, 8)])
            pltpu.sync_copy(zbuf2, den_sh.at[pl.ds(row0 + i * 8, 8)])
        plsc.subcore_barrier()

        def chunk(nc, _):
            base = wid * EPT + nc * C
            pltpu.sync_copy(src_hbm.at[pl.ds(base, C)], sidx)
            pltpu.sync_copy(dst_hbm.at[pl.ds(base, C)], didx)
            pltpu.sync_copy(logits_hbm.at[pl.ds(base, C)], lbuf)
            pltpu.async_copy(xt_hbm.at[sidx], U, sem).wait()

            def group(g, _):
                dvec = didx[pl.ds(g * 16, 16)]
                mv = plsc.load_gather(m_loc, [dvec])
                unn = jnp.exp(lbuf[pl.ds(g * 16, 16)] - mv)
                rows = g * 16 + iota
                plsc.store_scatter(den_b, [rows, jnp.zeros((16,), jnp.int32)],
                                   unn)
                ubuf[pl.ds(0, 16)] = unn

                def edge(j2, _):
                    for p in range(2):
                        jl = j2 * 2 + p
                        e = g * 16 + jl
                        ub = plsc.load_gather(
                            ubuf, [jnp.full((16,), jl, jnp.int32)])
                        for k in range(KG):
                            U[e, pl.ds(k * 16, 16)] = (
                                U[e, pl.ds(k * 16, 16)] * ub)
                    return 0
                lax.fori_loop(0, 8, edge, 0)
                return 0
            lax.fori_loop(0, C // 16, group, 0)
            pltpu.sync_copy(U, acc_sh.at[didx], add=True)
            pltpu.sync_copy(den_b, den_sh.at[didx], add=True)
            return 0
        lax.fori_loop(0, N_CHUNKS, chunk, 0)

        plsc.subcore_barrier()
        pltpu.sync_copy(acc_sh.at[pl.ds(row0, NR)],
                        acc_hbm.at[cid, pl.ds(row0, NR)])
        pltpu.sync_copy(den_sh.at[pl.ds(row0, NR)],
                        den_hbm.at[cid, pl.ds(row0, NR)])

    return pl.kernel(
        body,
        out_type=[jax.ShapeDtypeStruct((2, N_PAD, D), jnp.float32),
                  jax.ShapeDtypeStruct((2, N_PAD, DEN_W), jnp.float32)],
        mesh=_mesh,
        compiler_params=_sc_params,
        scratch_types=[
            pltpu.VMEM((C,), jnp.int32),
            pltpu.VMEM((C,), jnp.int32),
            pltpu.VMEM((C,), jnp.float32),
            pltpu.VMEM((C, D), jnp.float32),
            pltpu.VMEM((C, DEN_W), jnp.float32),
            pltpu.VMEM((N_PAD,), jnp.float32),
            pltpu.VMEM((8, D), jnp.float32),
            pltpu.VMEM((8, DEN_W), jnp.float32),
            pltpu.VMEM((16,), jnp.float32),
            pltpu.VMEM_SHARED((N_PAD, D), jnp.float32),
            pltpu.VMEM_SHARED((N_PAD, DEN_W), jnp.float32),
            pltpu.SemaphoreType.DMA,
        ],
    )


_phase1_128 = _make_phase1(128)
_phase2_64 = _make_phase2(64)
_phase1_16 = _make_phase1(16)
_phase2_16 = _make_phase2(16)

_BLK = 1024


def _tc_in(x_pad, wl, wr):
    def body(x_ref, wl_ref, wr_ref, xl_ref, xr_ref, xlo_ref, xhi_ref):
        xb = x_ref[...]
        xl = jnp.dot(xb, wl_ref[...], preferred_element_type=jnp.float32)
        xl_ref[...] = xl
        xr_ref[...] = jnp.dot(xb, wr_ref[...],
                              preferred_element_type=jnp.float32)
        xlo_ref[...] = xl[:, :64]
        xhi_ref[...] = xl[:, 64:]
    return pl.pallas_call(
        body,
        grid=(N_PAD // _BLK,),
        in_specs=[pl.BlockSpec((_BLK, F_IN), lambda i: (i, 0)),
                  pl.BlockSpec((F_IN, HID), lambda i: (0, 0)),
                  pl.BlockSpec((F_IN, HID), lambda i: (0, 0))],
        out_specs=[pl.BlockSpec((_BLK, HID), lambda i: (i, 0)),
                   pl.BlockSpec((_BLK, HID), lambda i: (i, 0)),
                   pl.BlockSpec((_BLK, 64), lambda i: (i, 0)),
                   pl.BlockSpec((_BLK, 64), lambda i: (i, 0))],
        out_shape=[jax.ShapeDtypeStruct((N_PAD, HID), jnp.float32),
                   jax.ShapeDtypeStruct((N_PAD, HID), jnp.float32),
                   jax.ShapeDtypeStruct((N_PAD, 64), jnp.float32),
                   jax.ShapeDtypeStruct((N_PAD, 64), jnp.float32)],
    )(x_pad, wl, wr)


def _tc_maxmerge(m):
    def body(m_ref, o_ref):
        o_ref[...] = jnp.maximum(m_ref[0], m_ref[1])
    return pl.pallas_call(
        body,
        grid=(N_PAD // _BLK,),
        in_specs=[pl.BlockSpec((2, _BLK), lambda i: (0, i))],
        out_specs=[pl.BlockSpec((_BLK,), lambda i: (i,))],
        out_shape=[jax.ShapeDtypeStruct((N_PAD,), jnp.float32)],
    )(m)[0]


def _tc_mid(acc_lo, acc_hi, den, b1, w2l, w2r):
    def body(alo_ref, ahi_ref, den_ref, b_ref, wl_ref, wr_ref,
             xl_ref, xr_ref):
        s = jnp.concatenate([alo_ref[0] + alo_ref[1],
                             ahi_ref[0] + ahi_ref[1]], axis=1)
        d = den_ref[0, :, 0:1] + den_ref[1, :, 0:1]
        h = s / (d + 1e-16) + b_ref[...]
        h = jnp.where(h > 0, h, jnp.exp(h) - 1.0)
        xl_ref[...] = jnp.dot(h, wl_ref[...],
                              preferred_element_type=jnp.float32)
        xr_ref[...] = jnp.dot(h, wr_ref[...],
                              preferred_element_type=jnp.float32)
    return pl.pallas_call(
        body,
        grid=(N_PAD // _BLK,),
        in_specs=[pl.BlockSpec((2, _BLK, 64), lambda i: (0, i, 0)),
                  pl.BlockSpec((2, _BLK, 64), lambda i: (0, i, 0)),
                  pl.BlockSpec((2, _BLK, DEN_W), lambda i: (0, i, 0)),
                  pl.BlockSpec((1, HID), lambda i: (0, 0)),
                  pl.BlockSpec((HID, NC), lambda i: (0, 0)),
                  pl.BlockSpec((HID, NC), lambda i: (0, 0))],
        out_specs=[pl.BlockSpec((_BLK, NC), lambda i: (i, 0)),
                   pl.BlockSpec((_BLK, NC), lambda i: (i, 0))],
        out_shape=[jax.ShapeDtypeStruct((N_PAD, NC), jnp.float32),
                   jax.ShapeDtypeStruct((N_PAD, NC), jnp.float32)],
    )(acc_lo, acc_hi, den, b1, w2l, w2r)


def _tc_out(acc, den, b2):
    def body(acc_ref, den_ref, b_ref, h_ref, ls_ref):
        s = acc_ref[0] + acc_ref[1]
        d = den_ref[0, :, 0:1] + den_ref[1, :, 0:1]
        h = s / (d + 1e-16) + b_ref[...]
        h_ref[...] = h
        m = jnp.max(h, axis=1, keepdims=True)
        ls_ref[...] = (h - m) - jnp.log(
            jnp.sum(jnp.exp(h - m), axis=1, keepdims=True))
    return pl.pallas_call(
        body,
        grid=(N_PAD // _BLK,),
        in_specs=[pl.BlockSpec((2, _BLK, NC), lambda i: (0, i, 0)),
                  pl.BlockSpec((2, _BLK, DEN_W), lambda i: (0, i, 0)),
                  pl.BlockSpec((1, NC), lambda i: (0, 0))],
        out_specs=[pl.BlockSpec((_BLK, NC), lambda i: (i, 0)),
                   pl.BlockSpec((_BLK, NC), lambda i: (i, 0))],
        out_shape=[jax.ShapeDtypeStruct((N_PAD, NC), jnp.float32),
                   jax.ShapeDtypeStruct((N_PAD, NC), jnp.float32)],
    )(acc, den, b2)


def kernel(x, edge_index, W1l, W1r, a1, b1, W2l, W2r, a2, b2):
    n = x.shape[0]
    i32 = jnp.int32
    loops = jnp.arange(n, dtype=i32)
    fill = jnp.full((E_PAD - E - n,), n, dtype=i32)
    src = jnp.concatenate([edge_index[0].astype(i32), loops, fill])
    dst = jnp.concatenate([edge_index[1].astype(i32), loops, fill])
    x_pad = jnp.zeros((N_PAD, F_IN), jnp.float32).at[:n].set(x)

    xl1, xr1, xl1_lo, xl1_hi = _tc_in(x_pad, W1l, W1r)
    logits1, m1, _mp1 = _phase1_128(xl1, xr1, src, dst, a1)
    mg1 = _tc_maxmerge(m1)
    acc_lo, den1 = _phase2_64(xl1_lo, src, dst, logits1, mg1)
    acc_hi, _den_u = _phase2_64(xl1_hi, src, dst, logits1, mg1)
    xl2, xr2 = _tc_mid(acc_lo, acc_hi, den1, b1.reshape(1, HID), W2l, W2r)
    logits2, m2, _mp2 = _phase1_16(xl2, xr2, src, dst, a2)
    mg2 = _tc_maxmerge(m2)
    acc2, den2 = _phase2_16(xl2, src, dst, logits2, mg2)
    h2, ls = _tc_out(acc2, den2, b2.reshape(1, NC))
    return (h2[:n], ls[:n])


# speedup vs baseline: 1.7469x; 1.7469x over previous
import jax
import jax.numpy as jnp
from jax import lax
from jax.experimental import pallas as pl
from jax.experimental.pallas import tpu as pltpu
from jax.experimental.pallas import tpu_sc as plsc

N = 10000
E = 320000
F_IN = 128
HID = 128
NC = 16

N_PAD = 10240
NR = N_PAD // 16
C = 128
E_TOT = E + N
N_CHUNKS = 82
EPT = N_CHUNKS * C
E_PAD = 32 * EPT
DEN_W = 16

_mesh = plsc.VectorSubcoreMesh(core_axis_name="c", subcore_axis_name="s",
                               num_cores=2, num_subcores=16)
_sc_params = pltpu.CompilerParams(needs_layout_passes=False,
                                  use_tc_tiling_on_sc=False)


def _seg_max_update(m_loc, dvec, lg):
    def cond(state):
        it, go = state
        return jnp.logical_and(go, it < 16)

    def step(state):
        it, _ = state
        cur = plsc.load_gather(m_loc, [dvec])
        need = lg > cur
        plsc.store_scatter(m_loc, [dvec], jnp.maximum(cur, lg), mask=need)
        cur2 = plsc.load_gather(m_loc, [dvec])
        return it + 1, jnp.any(lg > cur2)

    lax.while_loop(cond, step, (jnp.int32(0), jnp.bool_(True)))


def _make_phase1(D):
    KG = D // 16

    def body(xl_hbm, xr_hbm, src_hbm, dst_hbm, a_hbm,
             logits_hbm, m_hbm, mpart_hbm,
             a_v, sidx0, sidx1, didx0, didx1, U0, U1, V0, V1,
             lbuf, m_loc, mbuf, mred,
             sem_i0, sem_i1, sem_r0, sem_r1):
        sidx = (sidx0, sidx1)
        didx = (didx0, didx1)
        U = (U0, U1)
        V = (V0, V1)
        sem_i = (sem_i0, sem_i1)
        sem_r = (sem_r0, sem_r1)
        cid = lax.axis_index("c")
        tid = lax.axis_index("s")
        wid = cid * 16 + tid
        iota = lax.iota(jnp.int32, 16)
        pltpu.sync_copy(a_hbm, a_v)
        neg = jnp.full((16,), -3e38, jnp.float32)

        def zi(i, _):
            m_loc[pl.ds(i * 16, 16)] = neg
            return 0
        lax.fori_loop(0, N_PAD // 16, zi, 0)

        def start_idx(t, p):
            base = wid * EPT + t * C
            pltpu.async_copy(src_hbm.at[pl.ds(base, C)], sidx[p], sem_i[p])
            pltpu.async_copy(dst_hbm.at[pl.ds(base, C)], didx[p], sem_i[p])

        def wait_idx(t, p):
            base = wid * EPT + t * C
            pltpu.make_async_copy(src_hbm.at[pl.ds(base, C)], sidx[p],
                                  sem_i[p]).wait()
            pltpu.make_async_copy(dst_hbm.at[pl.ds(base, C)], didx[p],
                                  sem_i[p]).wait()

        def start_row(p):
            pltpu.async_copy(xl_hbm.at[sidx[p]], U[p], sem_r[p])
            pltpu.async_copy(xr_hbm.at[didx[p]], V[p], sem_r[p])

        def wait_row(p):
            pltpu.make_async_copy(xl_hbm.at[sidx[p]], U[p], sem_r[p]).wait()
            pltpu.make_async_copy(xr_hbm.at[didx[p]], V[p], sem_r[p]).wait()

        def compute(t, p):
            Ub, Vb, db = U[p], V[p], didx[p]
            base = wid * EPT + t * C

            def group(g, _):
                def edge(j, lg):
                    e = g * 16 + j
                    acc = jnp.zeros((16,), jnp.float32)
                    for k in range(KG):
                        u = Ub[e, pl.ds(k * 16, 16)]
                        v = Vb[e, pl.ds(k * 16, 16)]
                        z = u + v
                        tt = jnp.maximum(z, 0.2 * z)
                        acc = acc + tt * a_v[pl.ds(k * 16, 16)]
                    s = jnp.sum(acc)
                    return jnp.where(iota == j, s, lg)
                lg = lax.fori_loop(0, 16, edge, jnp.zeros((16,), jnp.float32))
                lbuf[pl.ds(g * 16, 16)] = lg
                dvec = db[pl.ds(g * 16, 16)]
                _seg_max_update(m_loc, dvec, lg)
                return 0
            lax.fori_loop(0, C // 16, group, 0)
            pltpu.sync_copy(lbuf, logits_hbm.at[pl.ds(base, C)])

        start_idx(0, 0)
        wait_idx(0, 0)
        start_row(0)
        start_idx(1, 1)

        def pair(t2, _):
            t = 2 * t2 + 1
            wait_idx(t, 1)
            start_row(1)
            wait_row(0)
            compute(t - 1, 0)
            start_idx(t + 1, 0)
            wait_idx(t + 1, 0)
            start_row(0)
            wait_row(1)
            compute(t, 1)
            start_idx(t + 2, 1)
            return 0
        lax.fori_loop(0, (N_CHUNKS - 2) // 2, pair, 0)
        t_last = N_CHUNKS - 1
        wait_idx(t_last, 1)
        start_row(1)
        wait_row(0)
        compute(t_last - 1, 0)
        wait_row(1)
        compute(t_last, 1)

        pltpu.sync_copy(m_loc, mpart_hbm.at[wid])
        plsc.subcore_barrier()
        row0 = tid * NR
        pltpu.sync_copy(mpart_hbm.at[pl.ds(cid * 16, 16), pl.ds(row0, NR)],
                        mbuf)

        def red(i, _):
            mv = mbuf[0, pl.ds(i * 16, 16)]
            for t in range(1, 16):
                mv = jnp.maximum(mv, mbuf[t, pl.ds(i * 16, 16)])
            mred[pl.ds(i * 16, 16)] = mv
            return 0
        lax.fori_loop(0, NR // 16, red, 0)
        pltpu.sync_copy(mred, m_hbm.at[cid, pl.ds(row0, NR)])

    return pl.kernel(
        body,
        out_type=[jax.ShapeDtypeStruct((E_PAD,), jnp.float32),
                  jax.ShapeDtypeStruct((2, N_PAD), jnp.float32),
                  jax.ShapeDtypeStruct((32, N_PAD), jnp.float32)],
        mesh=_mesh,
        compiler_params=_sc_params,
        scratch_types=[
            pltpu.VMEM((D,), jnp.float32),
            pltpu.VMEM((C,), jnp.int32),
            pltpu.VMEM((C,), jnp.int32),
            pltpu.VMEM((C,), jnp.int32),
            pltpu.VMEM((C,), jnp.int32),
            pltpu.VMEM((C, D), jnp.float32),
            pltpu.VMEM((C, D), jnp.float32),
            pltpu.VMEM((C, D), jnp.float32),
            pltpu.VMEM((C, D), jnp.float32),
            pltpu.VMEM((C,), jnp.float32),
            pltpu.VMEM((N_PAD,), jnp.float32),
            pltpu.VMEM((16, NR), jnp.float32),
            pltpu.VMEM((NR,), jnp.float32),
            pltpu.SemaphoreType.DMA,
            pltpu.SemaphoreType.DMA,
            pltpu.SemaphoreType.DMA,
            pltpu.SemaphoreType.DMA,
        ],
    )


def _make_phase2(D):
    KG = D // 16

    def body(xt_hbm, src_hbm, dst_hbm, logits_hbm, mg_hbm,
             acc_hbm, den_hbm,
             sidx0, sidx1, didx0, didx1, lbuf0, lbuf1, U0, U1,
             den_b, m_loc, zbuf, zbuf2,
             sem_i0, sem_i1, sem_r0, sem_r1,
             acc_sh, den_sh):
        sidx = (sidx0, sidx1)
        didx = (didx0, didx1)
        lbufs = (lbuf0, lbuf1)
        U = (U0, U1)
        sem_i = (sem_i0, sem_i1)
        sem_r = (sem_r0, sem_r1)
        cid = lax.axis_index("c")
        tid = lax.axis_index("s")
        wid = cid * 16 + tid
        iota = lax.iota(jnp.int32, 16)
        zeros16 = jnp.zeros((16,), jnp.float32)

        pltpu.sync_copy(mg_hbm, m_loc)

        def zb(r, _):
            for k in range(KG):
                zbuf[r, pl.ds(k * 16, 16)] = zeros16
            zbuf2[r, pl.ds(0, 16)] = zeros16
            return 0
        lax.fori_loop(0, 8, zb, 0)

        def zd(r, _):
            den_b[r, pl.ds(0, 16)] = zeros16
            return 0
        lax.fori_loop(0, C, zd, 0)
        row0 = tid * NR
        for i in range(NR // 8):
            pltpu.sync_copy(zbuf, acc_sh.at[pl.ds(row0 + i * 8, 8)])
            pltpu.sync_copy(zbuf2, den_sh.at[pl.ds(row0 + i * 8, 8)])
        plsc.subcore_barrier()

        def start_idx(t, p):
            base = wid * EPT + t * C
            pltpu.async_copy(src_hbm.at[pl.ds(base, C)], sidx[p], sem_i[p])
            pltpu.async_copy(dst_hbm.at[pl.ds(base, C)], didx[p], sem_i[p])
            pltpu.async_copy(logits_hbm.at[pl.ds(base, C)], lbufs[p],
                             sem_i[p])

        def wait_idx(t, p):
            base = wid * EPT + t * C
            pltpu.make_async_copy(src_hbm.at[pl.ds(base, C)], sidx[p],
                                  sem_i[p]).wait()
            pltpu.make_async_copy(dst_hbm.at[pl.ds(base, C)], didx[p],
                                  sem_i[p]).wait()
            pltpu.make_async_copy(logits_hbm.at[pl.ds(base, C)], lbufs[p],
                                  sem_i[p]).wait()

        def start_row(p):
            pltpu.async_copy(xt_hbm.at[sidx[p]], U[p], sem_r[p])

        def wait_row(p):
            pltpu.make_async_copy(xt_hbm.at[sidx[p]], U[p], sem_r[p]).wait()

        def compute(t, p):
            Ub, db, lb = U[p], didx[p], lbufs[p]

            def group(g, _):
                dvec = db[pl.ds(g * 16, 16)]
                mv = plsc.load_gather(m_loc, [dvec])
                unn = jnp.exp(lb[pl.ds(g * 16, 16)] - mv)
                rows = g * 16 + iota
                plsc.store_scatter(den_b, [rows, jnp.zeros((16,), jnp.int32)],
                                   unn)

                def edge(j, _):
                    s = jnp.sum(jnp.where(iota == j, unn, 0.0))
                    e = g * 16 + j
                    for k in range(KG):
                        Ub[e, pl.ds(k * 16, 16)] = (
                            Ub[e, pl.ds(k * 16, 16)] * s)
                    return 0
                lax.fori_loop(0, 16, edge, 0)
                return 0
            lax.fori_loop(0, C // 16, group, 0)
            pltpu.sync_copy(U[p], acc_sh.at[didx[p]], add=True)
            pltpu.sync_copy(den_b, den_sh.at[didx[p]], add=True)

        start_idx(0, 0)
        wait_idx(0, 0)
        start_row(0)
        start_idx(1, 1)

        def pair(t2, _):
            t = 2 * t2 + 1
            wait_idx(t, 1)
            start_row(1)
            wait_row(0)
            compute(t - 1, 0)
            start_idx(t + 1, 0)
            wait_idx(t + 1, 0)
            start_row(0)
            wait_row(1)
            compute(t, 1)
            start_idx(t + 2, 1)
            return 0
        lax.fori_loop(0, (N_CHUNKS - 2) // 2, pair, 0)
        t_last = N_CHUNKS - 1
        wait_idx(t_last, 1)
        start_row(1)
        wait_row(0)
        compute(t_last - 1, 0)
        wait_row(1)
        compute(t_last, 1)

        plsc.subcore_barrier()
        pltpu.sync_copy(acc_sh.at[pl.ds(row0, NR)],
                        acc_hbm.at[cid, pl.ds(row0, NR)])
        pltpu.sync_copy(den_sh.at[pl.ds(row0, NR)],
                        den_hbm.at[cid, pl.ds(row0, NR)])

    return pl.kernel(
        body,
        out_type=[jax.ShapeDtypeStruct((2, N_PAD, D), jnp.float32),
                  jax.ShapeDtypeStruct((2, N_PAD, DEN_W), jnp.float32)],
        mesh=_mesh,
        compiler_params=_sc_params,
        scratch_types=[
            pltpu.VMEM((C,), jnp.int32),
            pltpu.VMEM((C,), jnp.int32),
            pltpu.VMEM((C,), jnp.int32),
            pltpu.VMEM((C,), jnp.int32),
            pltpu.VMEM((C,), jnp.float32),
            pltpu.VMEM((C,), jnp.float32),
            pltpu.VMEM((C, D), jnp.float32),
            pltpu.VMEM((C, D), jnp.float32),
            pltpu.VMEM((C, DEN_W), jnp.float32),
            pltpu.VMEM((N_PAD,), jnp.float32),
            pltpu.VMEM((8, D), jnp.float32),
            pltpu.VMEM((8, DEN_W), jnp.float32),
            pltpu.SemaphoreType.DMA,
            pltpu.SemaphoreType.DMA,
            pltpu.SemaphoreType.DMA,
            pltpu.SemaphoreType.DMA,
            pltpu.VMEM_SHARED((N_PAD, D), jnp.float32),
            pltpu.VMEM_SHARED((N_PAD, DEN_W), jnp.float32),
        ],
    )


_phase1_128 = _make_phase1(128)
_phase2_64 = _make_phase2(64)
_phase1_16 = _make_phase1(16)
_phase2_16 = _make_phase2(16)

_BLK = 1024


def _tc_in(x_pad, wl, wr):
    def body(x_ref, wl_ref, wr_ref, xl_ref, xr_ref, xlo_ref, xhi_ref):
        xb = x_ref[...]
        xl = jnp.dot(xb, wl_ref[...], preferred_element_type=jnp.float32)
        xl_ref[...] = xl
        xr_ref[...] = jnp.dot(xb, wr_ref[...],
                              preferred_element_type=jnp.float32)
        xlo_ref[...] = xl[:, :64]
        xhi_ref[...] = xl[:, 64:]
    return pl.pallas_call(
        body,
        grid=(N_PAD // _BLK,),
        in_specs=[pl.BlockSpec((_BLK, F_IN), lambda i: (i, 0)),
                  pl.BlockSpec((F_IN, HID), lambda i: (0, 0)),
                  pl.BlockSpec((F_IN, HID), lambda i: (0, 0))],
        out_specs=[pl.BlockSpec((_BLK, HID), lambda i: (i, 0)),
                   pl.BlockSpec((_BLK, HID), lambda i: (i, 0)),
                   pl.BlockSpec((_BLK, 64), lambda i: (i, 0)),
                   pl.BlockSpec((_BLK, 64), lambda i: (i, 0))],
        out_shape=[jax.ShapeDtypeStruct((N_PAD, HID), jnp.float32),
                   jax.ShapeDtypeStruct((N_PAD, HID), jnp.float32),
                   jax.ShapeDtypeStruct((N_PAD, 64), jnp.float32),
                   jax.ShapeDtypeStruct((N_PAD, 64), jnp.float32)],
    )(x_pad, wl, wr)


def _tc_maxmerge(m):
    def body(m_ref, o_ref):
        o_ref[...] = jnp.maximum(m_ref[0], m_ref[1])
    return pl.pallas_call(
        body,
        grid=(N_PAD // _BLK,),
        in_specs=[pl.BlockSpec((2, _BLK), lambda i: (0, i))],
        out_specs=[pl.BlockSpec((_BLK,), lambda i: (i,))],
        out_shape=[jax.ShapeDtypeStruct((N_PAD,), jnp.float32)],
    )(m)[0]


def _tc_mid(acc_lo, acc_hi, den, b1, w2l, w2r):
    def body(alo_ref, ahi_ref, den_ref, b_ref, wl_ref, wr_ref,
             xl_ref, xr_ref):
        s = jnp.concatenate([alo_ref[0] + alo_ref[1],
                             ahi_ref[0] + ahi_ref[1]], axis=1)
        d = den_ref[0, :, 0:1] + den_ref[1, :, 0:1]
        h = s / (d + 1e-16) + b_ref[...]
        h = jnp.where(h > 0, h, jnp.exp(h) - 1.0)
        xl_ref[...] = jnp.dot(h, wl_ref[...],
                              preferred_element_type=jnp.float32)
        xr_ref[...] = jnp.dot(h, wr_ref[...],
                              preferred_element_type=jnp.float32)
    return pl.pallas_call(
        body,
        grid=(N_PAD // _BLK,),
        in_specs=[pl.BlockSpec((2, _BLK, 64), lambda i: (0, i, 0)),
                  pl.BlockSpec((2, _BLK, 64), lambda i: (0, i, 0)),
                  pl.BlockSpec((2, _BLK, DEN_W), lambda i: (0, i, 0)),
                  pl.BlockSpec((1, HID), lambda i: (0, 0)),
                  pl.BlockSpec((HID, NC), lambda i: (0, 0)),
                  pl.BlockSpec((HID, NC), lambda i: (0, 0))],
        out_specs=[pl.BlockSpec((_BLK, NC), lambda i: (i, 0)),
                   pl.BlockSpec((_BLK, NC), lambda i: (i, 0))],
        out_shape=[jax.ShapeDtypeStruct((N_PAD, NC), jnp.float32),
                   jax.ShapeDtypeStruct((N_PAD, NC), jnp.float32)],
    )(acc_lo, acc_hi, den, b1, w2l, w2r)


def _tc_out(acc, den, b2):
    def body(acc_ref, den_ref, b_ref, h_ref, ls_ref):
        s = acc_ref[0] + acc_ref[1]
        d = den_ref[0, :, 0:1] + den_ref[1, :, 0:1]
        h = s / (d + 1e-16) + b_ref[...]
        h_ref[...] = h
        m = jnp.max(h, axis=1, keepdims=True)
        ls_ref[...] = (h - m) - jnp.log(
            jnp.sum(jnp.exp(h - m), axis=1, keepdims=True))
    return pl.pallas_call(
        body,
        grid=(N_PAD // _BLK,),
        in_specs=[pl.BlockSpec((2, _BLK, NC), lambda i: (0, i, 0)),
                  pl.BlockSpec((2, _BLK, DEN_W), lambda i: (0, i, 0)),
                  pl.BlockSpec((1, NC), lambda i: (0, 0))],
        out_specs=[pl.BlockSpec((_BLK, NC), lambda i: (i, 0)),
                   pl.BlockSpec((_BLK, NC), lambda i: (i, 0))],
        out_shape=[jax.ShapeDtypeStruct((N_PAD, NC), jnp.float32),
                   jax.ShapeDtypeStruct((N_PAD, NC), jnp.float32)],
    )(acc, den, b2)


def kernel(x, edge_index, W1l, W1r, a1, b1, W2l, W2r, a2, b2):
    n = x.shape[0]
    i32 = jnp.int32
    loops = jnp.arange(n, dtype=i32)
    fill = jnp.full((E_PAD - E - n,), n, dtype=i32)
    src = jnp.concatenate([edge_index[0].astype(i32), loops, fill])
    dst = jnp.concatenate([edge_index[1].astype(i32), loops, fill])
    x_pad = jnp.zeros((N_PAD, F_IN), jnp.float32).at[:n].set(x)

    xl1, xr1, xl1_lo, xl1_hi = _tc_in(x_pad, W1l, W1r)
    logits1, m1, _mp1 = _phase1_128(xl1, xr1, src, dst, a1)
    mg1 = _tc_maxmerge(m1)
    acc_lo, den1 = _phase2_64(xl1_lo, src, dst, logits1, mg1)
    acc_hi, _den_u = _phase2_64(xl1_hi, src, dst, logits1, mg1)
    xl2, xr2 = _tc_mid(acc_lo, acc_hi, den1, b1.reshape(1, HID), W2l, W2r)
    logits2, m2, _mp2 = _phase1_16(xl2, xr2, src, dst, a2)
    mg2 = _tc_maxmerge(m2)
    acc2, den2 = _phase2_16(xl2, src, dst, logits2, mg2)
    h2, ls = _tc_out(acc2, den2, b2.reshape(1, NC))
    return (h2[:n], ls[:n])


# speedup vs baseline: 1.8117x; 1.0371x over previous
import jax
import jax.numpy as jnp
from jax import lax
from jax.experimental import pallas as pl
from jax.experimental.pallas import tpu as pltpu
from jax.experimental.pallas import tpu_sc as plsc

N = 10000
E = 320000
F_IN = 128
HID = 128
NC = 16

N_PAD = 10240
NR = N_PAD // 16
C = 128
E_TOT = E + N
N_CHUNKS = 82
EPT = N_CHUNKS * C
E_PAD = 32 * EPT
DEN_W = 16

_mesh = plsc.VectorSubcoreMesh(core_axis_name="c", subcore_axis_name="s",
                               num_cores=2, num_subcores=16)
_sc_params = pltpu.CompilerParams(needs_layout_passes=False,
                                  use_tc_tiling_on_sc=False)


def _seg_max_update(m_loc, dvec, lg):
    def cond(state):
        it, go = state
        return jnp.logical_and(go, it < 16)

    def step(state):
        it, _ = state
        cur = plsc.load_gather(m_loc, [dvec])
        need = lg > cur
        plsc.store_scatter(m_loc, [dvec], jnp.maximum(cur, lg), mask=need)
        cur2 = plsc.load_gather(m_loc, [dvec])
        return it + 1, jnp.any(lg > cur2)

    lax.while_loop(cond, step, (jnp.int32(0), jnp.bool_(True)))


def _make_phase1(D):
    KG = D // 16

    def body(xl_hbm, xr_hbm, src_hbm, dst_hbm, a_hbm,
             logits_hbm, m_hbm, mpart_hbm,
             a_v, sidx0, sidx1, didx0, didx1, U0, U1, V0, V1,
             lbuf, m_loc, mbuf, mred,
             sem_i0, sem_i1, sem_r0, sem_r1):
        sidx = (sidx0, sidx1)
        didx = (didx0, didx1)
        U = (U0, U1)
        V = (V0, V1)
        sem_i = (sem_i0, sem_i1)
        sem_r = (sem_r0, sem_r1)
        cid = lax.axis_index("c")
        tid = lax.axis_index("s")
        wid = cid * 16 + tid
        iota = lax.iota(jnp.int32, 16)
        pltpu.sync_copy(a_hbm, a_v)
        neg = jnp.full((16,), -3e38, jnp.float32)

        def zi(i, _):
            m_loc[pl.ds(i * 16, 16)] = neg
            return 0
        lax.fori_loop(0, N_PAD // 16, zi, 0)

        def start_idx(t, p):
            base = wid * EPT + t * C
            pltpu.async_copy(src_hbm.at[pl.ds(base, C)], sidx[p], sem_i[p])
            pltpu.async_copy(dst_hbm.at[pl.ds(base, C)], didx[p], sem_i[p])

        def wait_idx(t, p):
            base = wid * EPT + t * C
            pltpu.make_async_copy(src_hbm.at[pl.ds(base, C)], sidx[p],
                                  sem_i[p]).wait()
            pltpu.make_async_copy(dst_hbm.at[pl.ds(base, C)], didx[p],
                                  sem_i[p]).wait()

        def start_row(p):
            pltpu.async_copy(xl_hbm.at[sidx[p]], U[p], sem_r[p])
            pltpu.async_copy(xr_hbm.at[didx[p]], V[p], sem_r[p])

        def wait_row(p):
            pltpu.make_async_copy(xl_hbm.at[sidx[p]], U[p], sem_r[p]).wait()
            pltpu.make_async_copy(xr_hbm.at[didx[p]], V[p], sem_r[p]).wait()

        def compute(t, p):
            Ub, Vb, db = U[p], V[p], didx[p]
            base = wid * EPT + t * C

            def group(g, _):
                def edge(j2, lg):
                    for q in range(2):
                        j = j2 * 2 + q
                        e = g * 16 + j
                        acc = jnp.zeros((16,), jnp.float32)
                        for k in range(KG):
                            u = Ub[e, pl.ds(k * 16, 16)]
                            v = Vb[e, pl.ds(k * 16, 16)]
                            z = u + v
                            tt = jnp.maximum(z, 0.2 * z)
                            acc = acc + tt * a_v[pl.ds(k * 16, 16)]
                        s = jnp.sum(acc)
                        lg = jnp.where(iota == j, s, lg)
                    return lg
                lg = lax.fori_loop(0, 8, edge, jnp.zeros((16,), jnp.float32))
                lbuf[pl.ds(g * 16, 16)] = lg
                dvec = db[pl.ds(g * 16, 16)]
                _seg_max_update(m_loc, dvec, lg)
                return 0
            lax.fori_loop(0, C // 16, group, 0)
            pltpu.sync_copy(lbuf, logits_hbm.at[pl.ds(base, C)])

        start_idx(0, 0)
        wait_idx(0, 0)
        start_row(0)
        start_idx(1, 1)

        def pair(t2, _):
            t = 2 * t2 + 1
            wait_idx(t, 1)
            start_row(1)
            wait_row(0)
            compute(t - 1, 0)
            start_idx(t + 1, 0)
            wait_idx(t + 1, 0)
            start_row(0)
            wait_row(1)
            compute(t, 1)
            start_idx(t + 2, 1)
            return 0
        lax.fori_loop(0, (N_CHUNKS - 2) // 2, pair, 0)
        t_last = N_CHUNKS - 1
        wait_idx(t_last, 1)
        start_row(1)
        wait_row(0)
        compute(t_last - 1, 0)
        wait_row(1)
        compute(t_last, 1)

        pltpu.sync_copy(m_loc, mpart_hbm.at[wid])
        plsc.subcore_barrier()
        row0 = tid * NR
        pltpu.sync_copy(mpart_hbm.at[pl.ds(cid * 16, 16), pl.ds(row0, NR)],
                        mbuf)

        def red(i, _):
            mv = mbuf[0, pl.ds(i * 16, 16)]
            for t in range(1, 16):
                mv = jnp.maximum(mv, mbuf[t, pl.ds(i * 16, 16)])
            mred[pl.ds(i * 16, 16)] = mv
            return 0
        lax.fori_loop(0, NR // 16, red, 0)
        pltpu.sync_copy(mred, m_hbm.at[cid, pl.ds(row0, NR)])

    return pl.kernel(
        body,
        out_type=[jax.ShapeDtypeStruct((E_PAD,), jnp.float32),
                  jax.ShapeDtypeStruct((2, N_PAD), jnp.float32),
                  jax.ShapeDtypeStruct((32, N_PAD), jnp.float32)],
        mesh=_mesh,
        compiler_params=_sc_params,
        scratch_types=[
            pltpu.VMEM((D,), jnp.float32),
            pltpu.VMEM((C,), jnp.int32),
            pltpu.VMEM((C,), jnp.int32),
            pltpu.VMEM((C,), jnp.int32),
            pltpu.VMEM((C,), jnp.int32),
            pltpu.VMEM((C, D), jnp.float32),
            pltpu.VMEM((C, D), jnp.float32),
            pltpu.VMEM((C, D), jnp.float32),
            pltpu.VMEM((C, D), jnp.float32),
            pltpu.VMEM((C,), jnp.float32),
            pltpu.VMEM((N_PAD,), jnp.float32),
            pltpu.VMEM((16, NR), jnp.float32),
            pltpu.VMEM((NR,), jnp.float32),
            pltpu.SemaphoreType.DMA,
            pltpu.SemaphoreType.DMA,
            pltpu.SemaphoreType.DMA,
            pltpu.SemaphoreType.DMA,
        ],
    )


def _make_phase2(D):
    KG = D // 16

    def body(xt_hbm, src_hbm, dst_hbm, logits_hbm, mg_hbm,
             acc_hbm, den_hbm,
             sidx0, sidx1, didx0, didx1, lbuf0, lbuf1, U0, U1,
             den_b, m_loc, zbuf, zbuf2,
             sem_i0, sem_i1, sem_r0, sem_r1,
             acc_sh, den_sh):
        sidx = (sidx0, sidx1)
        didx = (didx0, didx1)
        lbufs = (lbuf0, lbuf1)
        U = (U0, U1)
        sem_i = (sem_i0, sem_i1)
        sem_r = (sem_r0, sem_r1)
        cid = lax.axis_index("c")
        tid = lax.axis_index("s")
        wid = cid * 16 + tid
        iota = lax.iota(jnp.int32, 16)
        zeros16 = jnp.zeros((16,), jnp.float32)

        pltpu.sync_copy(mg_hbm, m_loc)

        def zb(r, _):
            for k in range(KG):
                zbuf[r, pl.ds(k * 16, 16)] = zeros16
            zbuf2[r, pl.ds(0, 16)] = zeros16
            return 0
        lax.fori_loop(0, 8, zb, 0)

        def zd(r, _):
            den_b[r, pl.ds(0, 16)] = zeros16
            return 0
        lax.fori_loop(0, C, zd, 0)
        row0 = tid * NR
        for i in range(NR // 8):
            pltpu.sync_copy(zbuf, acc_sh.at[pl.ds(row0 + i * 8, 8)])
            pltpu.sync_copy(zbuf2, den_sh.at[pl.ds(row0 + i * 8, 8)])
        plsc.subcore_barrier()

        def start_idx(t, p):
            base = wid * EPT + t * C
            pltpu.async_copy(src_hbm.at[pl.ds(base, C)], sidx[p], sem_i[p])
            pltpu.async_copy(dst_hbm.at[pl.ds(base, C)], didx[p], sem_i[p])
            pltpu.async_copy(logits_hbm.at[pl.ds(base, C)], lbufs[p],
                             sem_i[p])

        def wait_idx(t, p):
            base = wid * EPT + t * C
            pltpu.make_async_copy(src_hbm.at[pl.ds(base, C)], sidx[p],
                                  sem_i[p]).wait()
            pltpu.make_async_copy(dst_hbm.at[pl.ds(base, C)], didx[p],
                                  sem_i[p]).wait()
            pltpu.make_async_copy(logits_hbm.at[pl.ds(base, C)], lbufs[p],
                                  sem_i[p]).wait()

        def start_row(p):
            pltpu.async_copy(xt_hbm.at[sidx[p]], U[p], sem_r[p])

        def wait_row(p):
            pltpu.make_async_copy(xt_hbm.at[sidx[p]], U[p], sem_r[p]).wait()

        def compute(t, p):
            Ub, db, lb = U[p], didx[p], lbufs[p]

            def group(g, _):
                dvec = db[pl.ds(g * 16, 16)]
                mv = plsc.load_gather(m_loc, [dvec])
                unn = jnp.exp(lb[pl.ds(g * 16, 16)] - mv)
                rows = g * 16 + iota
                plsc.store_scatter(den_b, [rows, jnp.zeros((16,), jnp.int32)],
                                   unn)

                def edge(j2, _):
                    for q in range(2):
                        j = j2 * 2 + q
                        s = jnp.sum(jnp.where(iota == j, unn, 0.0))
                        e = g * 16 + j
                        for k in range(KG):
                            Ub[e, pl.ds(k * 16, 16)] = (
                                Ub[e, pl.ds(k * 16, 16)] * s)
                    return 0
                lax.fori_loop(0, 8, edge, 0)
                return 0
            lax.fori_loop(0, C // 16, group, 0)
            pltpu.sync_copy(U[p], acc_sh.at[didx[p]], add=True)
            pltpu.sync_copy(den_b, den_sh.at[didx[p]], add=True)

        start_idx(0, 0)
        wait_idx(0, 0)
        start_row(0)
        start_idx(1, 1)

        def pair(t2, _):
            t = 2 * t2 + 1
            wait_idx(t, 1)
            start_row(1)
            wait_row(0)
            compute(t - 1, 0)
            start_idx(t + 1, 0)
            wait_idx(t + 1, 0)
            start_row(0)
            wait_row(1)
            compute(t, 1)
            start_idx(t + 2, 1)
            return 0
        lax.fori_loop(0, (N_CHUNKS - 2) // 2, pair, 0)
        t_last = N_CHUNKS - 1
        wait_idx(t_last, 1)
        start_row(1)
        wait_row(0)
        compute(t_last - 1, 0)
        wait_row(1)
        compute(t_last, 1)

        plsc.subcore_barrier()
        pltpu.sync_copy(acc_sh.at[pl.ds(row0, NR)],
                        acc_hbm.at[cid, pl.ds(row0, NR)])
        pltpu.sync_copy(den_sh.at[pl.ds(row0, NR)],
                        den_hbm.at[cid, pl.ds(row0, NR)])

    return pl.kernel(
        body,
        out_type=[jax.ShapeDtypeStruct((2, N_PAD, D), jnp.float32),
                  jax.ShapeDtypeStruct((2, N_PAD, DEN_W), jnp.float32)],
        mesh=_mesh,
        compiler_params=_sc_params,
        scratch_types=[
            pltpu.VMEM((C,), jnp.int32),
            pltpu.VMEM((C,), jnp.int32),
            pltpu.VMEM((C,), jnp.int32),
            pltpu.VMEM((C,), jnp.int32),
            pltpu.VMEM((C,), jnp.float32),
            pltpu.VMEM((C,), jnp.float32),
            pltpu.VMEM((C, D), jnp.float32),
            pltpu.VMEM((C, D), jnp.float32),
            pltpu.VMEM((C, DEN_W), jnp.float32),
            pltpu.VMEM((N_PAD,), jnp.float32),
            pltpu.VMEM((8, D), jnp.float32),
            pltpu.VMEM((8, DEN_W), jnp.float32),
            pltpu.SemaphoreType.DMA,
            pltpu.SemaphoreType.DMA,
            pltpu.SemaphoreType.DMA,
            pltpu.SemaphoreType.DMA,
            pltpu.VMEM_SHARED((N_PAD, D), jnp.float32),
            pltpu.VMEM_SHARED((N_PAD, DEN_W), jnp.float32),
        ],
    )


_phase1_128 = _make_phase1(128)
_phase2_64 = _make_phase2(64)
_phase1_16 = _make_phase1(16)
_phase2_16 = _make_phase2(16)

_BLK = 1024


def _tc_in(x_pad, wl, wr):
    def body(x_ref, wl_ref, wr_ref, xl_ref, xr_ref, xlo_ref, xhi_ref):
        xb = x_ref[...]
        xl = jnp.dot(xb, wl_ref[...], preferred_element_type=jnp.float32)
        xl_ref[...] = xl
        xr_ref[...] = jnp.dot(xb, wr_ref[...],
                              preferred_element_type=jnp.float32)
        xlo_ref[...] = xl[:, :64]
        xhi_ref[...] = xl[:, 64:]
    return pl.pallas_call(
        body,
        grid=(N_PAD // _BLK,),
        in_specs=[pl.BlockSpec((_BLK, F_IN), lambda i: (i, 0)),
                  pl.BlockSpec((F_IN, HID), lambda i: (0, 0)),
                  pl.BlockSpec((F_IN, HID), lambda i: (0, 0))],
        out_specs=[pl.BlockSpec((_BLK, HID), lambda i: (i, 0)),
                   pl.BlockSpec((_BLK, HID), lambda i: (i, 0)),
                   pl.BlockSpec((_BLK, 64), lambda i: (i, 0)),
                   pl.BlockSpec((_BLK, 64), lambda i: (i, 0))],
        out_shape=[jax.ShapeDtypeStruct((N_PAD, HID), jnp.float32),
                   jax.ShapeDtypeStruct((N_PAD, HID), jnp.float32),
                   jax.ShapeDtypeStruct((N_PAD, 64), jnp.float32),
                   jax.ShapeDtypeStruct((N_PAD, 64), jnp.float32)],
    )(x_pad, wl, wr)


def _tc_maxmerge(m):
    def body(m_ref, o_ref):
        o_ref[...] = jnp.maximum(m_ref[0], m_ref[1])
    return pl.pallas_call(
        body,
        grid=(N_PAD // _BLK,),
        in_specs=[pl.BlockSpec((2, _BLK), lambda i: (0, i))],
        out_specs=[pl.BlockSpec((_BLK,), lambda i: (i,))],
        out_shape=[jax.ShapeDtypeStruct((N_PAD,), jnp.float32)],
    )(m)[0]


def _tc_mid(acc_lo, acc_hi, den, b1, w2l, w2r):
    def body(alo_ref, ahi_ref, den_ref, b_ref, wl_ref, wr_ref,
             xl_ref, xr_ref):
        s = jnp.concatenate([alo_ref[0] + alo_ref[1],
                             ahi_ref[0] + ahi_ref[1]], axis=1)
        d = den_ref[0, :, 0:1] + den_ref[1, :, 0:1]
        h = s / (d + 1e-16) + b_ref[...]
        h = jnp.where(h > 0, h, jnp.exp(h) - 1.0)
        xl_ref[...] = jnp.dot(h, wl_ref[...],
                              preferred_element_type=jnp.float32)
        xr_ref[...] = jnp.dot(h, wr_ref[...],
                              preferred_element_type=jnp.float32)
    return pl.pallas_call(
        body,
        grid=(N_PAD // _BLK,),
        in_specs=[pl.BlockSpec((2, _BLK, 64), lambda i: (0, i, 0)),
                  pl.BlockSpec((2, _BLK, 64), lambda i: (0, i, 0)),
                  pl.BlockSpec((2, _BLK, DEN_W), lambda i: (0, i, 0)),
                  pl.BlockSpec((1, HID), lambda i: (0, 0)),
                  pl.BlockSpec((HID, NC), lambda i: (0, 0)),
                  pl.BlockSpec((HID, NC), lambda i: (0, 0))],
        out_specs=[pl.BlockSpec((_BLK, NC), lambda i: (i, 0)),
                   pl.BlockSpec((_BLK, NC), lambda i: (i, 0))],
        out_shape=[jax.ShapeDtypeStruct((N_PAD, NC), jnp.float32),
                   jax.ShapeDtypeStruct((N_PAD, NC), jnp.float32)],
    )(acc_lo, acc_hi, den, b1, w2l, w2r)


def _tc_out(acc, den, b2):
    def body(acc_ref, den_ref, b_ref, h_ref, ls_ref):
        s = acc_ref[0] + acc_ref[1]
        d = den_ref[0, :, 0:1] + den_ref[1, :, 0:1]
        h = s / (d + 1e-16) + b_ref[...]
        h_ref[...] = h
        m = jnp.max(h, axis=1, keepdims=True)
        ls_ref[...] = (h - m) - jnp.log(
            jnp.sum(jnp.exp(h - m), axis=1, keepdims=True))
    return pl.pallas_call(
        body,
        grid=(N_PAD // _BLK,),
        in_specs=[pl.BlockSpec((2, _BLK, NC), lambda i: (0, i, 0)),
                  pl.BlockSpec((2, _BLK, DEN_W), lambda i: (0, i, 0)),
                  pl.BlockSpec((1, NC), lambda i: (0, 0))],
        out_specs=[pl.BlockSpec((_BLK, NC), lambda i: (i, 0)),
                   pl.BlockSpec((_BLK, NC), lambda i: (i, 0))],
        out_shape=[jax.ShapeDtypeStruct((N_PAD, NC), jnp.float32),
                   jax.ShapeDtypeStruct((N_PAD, NC), jnp.float32)],
    )(acc, den, b2)


def kernel(x, edge_index, W1l, W1r, a1, b1, W2l, W2r, a2, b2):
    n = x.shape[0]
    i32 = jnp.int32
    loops = jnp.arange(n, dtype=i32)
    fill = jnp.full((E_PAD - E - n,), n, dtype=i32)
    src = jnp.concatenate([edge_index[0].astype(i32), loops, fill])
    dst = jnp.concatenate([edge_index[1].astype(i32), loops, fill])
    x_pad = jnp.zeros((N_PAD, F_IN), jnp.float32).at[:n].set(x)

    xl1, xr1, xl1_lo, xl1_hi = _tc_in(x_pad, W1l, W1r)
    logits1, m1, _mp1 = _phase1_128(xl1, xr1, src, dst, a1)
    mg1 = _tc_maxmerge(m1)
    acc_lo, den1 = _phase2_64(xl1_lo, src, dst, logits1, mg1)
    acc_hi, _den_u = _phase2_64(xl1_hi, src, dst, logits1, mg1)
    xl2, xr2 = _tc_mid(acc_lo, acc_hi, den1, b1.reshape(1, HID), W2l, W2r)
    logits2, m2, _mp2 = _phase1_16(xl2, xr2, src, dst, a2)
    mg2 = _tc_maxmerge(m2)
    acc2, den2 = _phase2_16(xl2, src, dst, logits2, mg2)
    h2, ls = _tc_out(acc2, den2, b2.reshape(1, NC))
    return (h2[:n], ls[:n])
